# Initial kernel scaffold; baseline (speedup 1.0000x reference)
import jax
import jax.numpy as jnp
from jax.experimental import pallas as pl


def kernel(x, edge_attr, additional_features, params, edge_index, batch):
    raise NotImplementedError("write your pallas kernel here")


# speedup vs baseline: 2.9751x; 2.9751x over previous
import functools

import jax
import jax.numpy as jnp
from jax import lax
from jax.experimental import pallas as pl
from jax.experimental.pallas import tpu as pltpu
from jax.experimental.pallas import tpu_sc as plsc

N = 50000
E = 800000
B = 256
H = 64
NHALF = N // 2
C = 128
NCC = 2
NSC = 16
NW = NCC * NSC
ECH = E // C
ECH_HALF = ECH // 2
NCH = N // C
NREM = N - NCH * C
HCH = NHALF // C
HREM = NHALF - HCH * C


@functools.cache
def _mesh():
    return plsc.VectorSubcoreMesh(
        core_axis_name="c", subcore_axis_name="s",
        num_cores=NCC, num_subcores=NSC)


_f32 = jnp.float32
_i32 = jnp.int32


def _sds(shape, dtype=_f32):
    return jax.ShapeDtypeStruct(shape, dtype)


def _prep_body(row_h, batch_h, cnt_h, ec_h, xc_h,
               idx_v, nbuf_v, ones_v, zb_v, ectab, xctab, cnt_sh):
    c = lax.axis_index("c")
    s = lax.axis_index("s")

    def init_ones(i, _):
        ones_v[i, :] = jnp.ones((16,), _f32)
        return 0
    lax.fori_loop(0, C, init_ones, 0)

    def init_zb(i, _):
        zb_v[i, :] = jnp.zeros((16,), _f32)
        return 0
    lax.fori_loop(0, 624, init_zb, 0)

    def init_tabs(i, _):
        ectab[i, :] = jnp.zeros((16,), _f32)
        xctab[i, :] = jnp.zeros((16,), _f32)
        return 0
    lax.fori_loop(0, B + 8, init_tabs, 0)

    def zloop(i, _):
        pltpu.sync_copy(zb_v, cnt_sh.at[pl.ds(s * 3120 + i * 624, 624), :])
        return 0
    lax.fori_loop(0, 5, zloop, 0)

    @pl.when(s == 0)
    def _():
        pltpu.sync_copy(zb_v.at[pl.ds(0, 80), :],
                        cnt_sh.at[pl.ds(NSC * 3120, 80), :])

    plsc.subcore_barrier()

    ne = 195 + jnp.where(s < 5, 1, 0)

    def ebody(i, _):
        cb = (c * ECH_HALF + s * 195 + jnp.minimum(s, 5) + i) * C
        pltpu.sync_copy(row_h.at[pl.ds(cb, C)], idx_v)
        pltpu.sync_copy(ones_v, cnt_sh.at[idx_v], add=True)
        return 0
    lax.fori_loop(0, ne, ebody, 0)

    plsc.subcore_barrier()

    def accgrp(g):
        bvec = idx_v[pl.ds(g * 16, 16)]
        for j in range(16):
            b = bvec[j]
            ectab[b, :] = ectab[b, :] + nbuf_v[g * 16 + j, :]
            xctab[b, :] = xctab[b, :] + jnp.ones((16,), _f32)

    nn = 24 + jnp.where(s < 6, 1, 0)

    def nbody(i, _):
        nb = (s * 24 + jnp.minimum(s, 6) + i) * C
        pltpu.sync_copy(batch_h.at[pl.ds(nb, C)], idx_v)
        pltpu.sync_copy(cnt_sh.at[pl.ds(nb, C), :], nbuf_v)

        def gb_(g, _):
            accgrp(g)
            return 0
        lax.fori_loop(0, C // 16, gb_, 0)
        return 0
    lax.fori_loop(0, nn, nbody, 0)

    @pl.when(s == NSC - 1)
    def _():
        nb = NCH * C
        pltpu.sync_copy(batch_h.at[pl.ds(nb, NREM)], idx_v.at[pl.ds(0, NREM)])
        pltpu.sync_copy(cnt_sh.at[pl.ds(nb, NREM), :],
                        nbuf_v.at[pl.ds(0, NREM), :])

        def gb_(g, _):
            accgrp(g)
            return 0
        lax.fori_loop(0, NREM // 16, gb_, 0)

    pltpu.sync_copy(cnt_sh.at[pl.ds(s * 3120, 3120), :],
                    cnt_h.at[c, pl.ds(s * 3120, 3120), :])

    @pl.when(s == 0)
    def _():
        pltpu.sync_copy(cnt_sh.at[pl.ds(NSC * 3120, 80), :],
                        cnt_h.at[c, pl.ds(NSC * 3120, 80), :])

    pltpu.sync_copy(ectab.at[pl.ds(0, B), :], ec_h.at[c, s])
    pltpu.sync_copy(xctab.at[pl.ds(0, B), :], xc_h.at[c, s])


@functools.cache
def _prep_k():
    return pl.kernel(
        _prep_body,
        out_type=(_sds((NCC, N, 16)), _sds((NCC, NSC, B, 16)),
                  _sds((NCC, NSC, B, 16))),
        mesh=_mesh(),
        compiler_params=pltpu.CompilerParams(use_tc_tiling_on_sc=False),
        scratch_types=[
            pltpu.VMEM((C,), _i32),
            pltpu.VMEM((C, 16), _f32),
            pltpu.VMEM((C, 16), _f32),
            pltpu.VMEM((624, 16), _f32),
            pltpu.VMEM((B + 8, 16), _f32),
            pltpu.VMEM((B + 8, 16), _f32),
            pltpu.VMEM_SHARED((N, 16), _f32),
        ])


def _gatherb_body(batch_h, ut_h, out_h, idx_v, rows_v, idx40_v, rows40_v, sem):
    c = lax.axis_index("c")
    s = lax.axis_index("s")
    w = s * NCC + c
    n = 12 + jnp.where(w < 6, 1, 0)

    def body(i, _):
        cb = (w * 12 + jnp.minimum(w, 6) + i) * C
        pltpu.sync_copy(batch_h.at[pl.ds(cb, C)], idx_v)
        pltpu.async_copy(ut_h.at[idx_v], rows_v, sem).wait()
        pltpu.sync_copy(rows_v, out_h.at[pl.ds(cb, C), :])
        return 0
    lax.fori_loop(0, n, body, 0)

    @pl.when(w == NW - 1)
    def _():
        pltpu.sync_copy(batch_h.at[pl.ds(NCH * C, NREM)], idx40_v)
        pltpu.async_copy(ut_h.at[idx40_v], rows40_v, sem).wait()
        pltpu.sync_copy(rows40_v, out_h.at[pl.ds(NCH * C, NREM), :])


@functools.cache
def _gatherb_k():
    return pl.kernel(
        _gatherb_body,
        out_type=_sds((N, 2 * H)),
        mesh=_mesh(),
        compiler_params=pltpu.CompilerParams(use_tc_tiling_on_sc=False),
        scratch_types=[
            pltpu.VMEM((C,), _i32), pltpu.VMEM((C, 2 * H), _f32),
            pltpu.VMEM((NREM,), _i32), pltpu.VMEM((NREM, 2 * H), _f32),
            pltpu.SemaphoreType.DMA,
        ])


def _gathere_body(row_h, col_h, t12_h, g_h,
                  ridx_v, cidx_v, a_v, b_v, g_buf, sem):
    c = lax.axis_index("c")
    s = lax.axis_index("s")
    w = s * NCC + c
    n = 195 + jnp.where(w < 10, 1, 0)

    def body(i, _):
        cb = (w * 195 + jnp.minimum(w, 10) + i) * C
        pltpu.sync_copy(row_h.at[pl.ds(cb, C)], ridx_v)
        pltpu.sync_copy(col_h.at[pl.ds(cb, C)], cidx_v)
        d1 = pltpu.async_copy(t12_h.at[ridx_v], a_v, sem)
        d2 = pltpu.async_copy(t12_h.at[cidx_v], b_v, sem)
        d1.wait()
        d2.wait()

        def abody(r, _):
            for q in range(4):
                sl = pl.ds(q * 16, 16)
                g_buf[r, sl] = a_v[r, sl] + b_v[r, pl.ds(H + q * 16, 16)]
            return 0
        lax.fori_loop(0, C, abody, 0)
        pltpu.sync_copy(g_buf, g_h.at[pl.ds(cb, C), :])
        return 0
    lax.fori_loop(0, n, body, 0)


@functools.cache
def _gathere_k():
    return pl.kernel(
        _gathere_body,
        out_type=_sds((E, H)),
        mesh=_mesh(),
        compiler_params=pltpu.CompilerParams(use_tc_tiling_on_sc=False),
        scratch_types=[
            pltpu.VMEM((C,), _i32), pltpu.VMEM((C,), _i32),
            pltpu.VMEM((C, 2 * H), _f32), pltpu.VMEM((C, 2 * H), _f32),
            pltpu.VMEM((C, H), _f32),
            pltpu.SemaphoreType.DMA,
        ])


def _scatter_body(enew_h, row_h, batch_h, agg_h, ep_h,
                  idx_v, e_buf, cbuf, nbuf_v, eptab, acc_sh):
    c = lax.axis_index("c")
    s = lax.axis_index("s")

    def init_cb(i, _):
        cbuf[i, pl.ds(0, 16)] = jnp.zeros((16,), _f32)
        cbuf[i, pl.ds(16, 16)] = jnp.zeros((16,), _f32)
        return 0
    lax.fori_loop(0, C, init_cb, 0)

    def init_tab(i, _):
        eptab[i, pl.ds(0, 16)] = jnp.zeros((16,), _f32)
        eptab[i, pl.ds(16, 16)] = jnp.zeros((16,), _f32)
        return 0
    lax.fori_loop(0, B + 8, init_tab, 0)

    def zloop(i, _):
        pltpu.sync_copy(cbuf, acc_sh.at[pl.ds(s * 3120 + i * C, C), :])
        return 0
    lax.fori_loop(0, 24, zloop, 0)

    pltpu.sync_copy(cbuf.at[pl.ds(0, 48), :],
                    acc_sh.at[pl.ds(s * 3120 + 24 * C, 48), :])

    @pl.when(s == 0)
    def _():
        pltpu.sync_copy(cbuf.at[pl.ds(0, 80), :],
                        acc_sh.at[pl.ds(NSC * 3120, 80), :])

    plsc.subcore_barrier()

    n = 390 + jnp.where(s < 10, 1, 0)

    def body(i, _):
        cb = (s * 390 + jnp.minimum(s, 10) + i) * C
        pltpu.sync_copy(enew_h.at[pl.ds(cb, C), :], e_buf)
        pltpu.sync_copy(row_h.at[pl.ds(cb, C)], idx_v)

        @pl.when(c == 0)
        def _():
            def cp0(r, _):
                cbuf[r, pl.ds(0, 16)] = e_buf[r, pl.ds(0, 16)]
                cbuf[r, pl.ds(16, 16)] = e_buf[r, pl.ds(16, 16)]
                return 0
            lax.fori_loop(0, C, cp0, 0)

        @pl.when(c == 1)
        def _():
            def cp1(r, _):
                cbuf[r, pl.ds(0, 16)] = e_buf[r, pl.ds(32, 16)]
                cbuf[r, pl.ds(16, 16)] = e_buf[r, pl.ds(48, 16)]
                return 0
            lax.fori_loop(0, C, cp1, 0)

        pltpu.sync_copy(cbuf, acc_sh.at[idx_v], add=True)
        return 0
    lax.fori_loop(0, n, body, 0)

    plsc.subcore_barrier()

    def accgrp(g):
        bvec = idx_v[pl.ds(g * 16, 16)]
        for j in range(16):
            b = bvec[j]
            for q in range(2):
                sl = pl.ds(q * 16, 16)
                eptab[b, sl] = eptab[b, sl] + nbuf_v[g * 16 + j, sl]

    nn = 24 + jnp.where(s < 6, 1, 0)

    def nbody(i, _):
        nb = (s * 24 + jnp.minimum(s, 6) + i) * C
        pltpu.sync_copy(batch_h.at[pl.ds(nb, C)], idx_v)
        pltpu.sync_copy(acc_sh.at[pl.ds(nb, C), :], nbuf_v)

        def gb_(g, _):
            accgrp(g)
            return 0
        lax.fori_loop(0, C // 16, gb_, 0)
        return 0
    lax.fori_loop(0, nn, nbody, 0)

    @pl.when(s == NSC - 1)
    def _():
        nb = NCH * C
        pltpu.sync_copy(batch_h.at[pl.ds(nb, NREM)], idx_v.at[pl.ds(0, NREM)])
        pltpu.sync_copy(acc_sh.at[pl.ds(nb, NREM), :],
                        nbuf_v.at[pl.ds(0, NREM), :])

        def gb_(g, _):
            accgrp(g)
            return 0
        lax.fori_loop(0, NREM // 16, gb_, 0)

    pltpu.sync_copy(acc_sh.at[pl.ds(s * 3120, 3120), :],
                    agg_h.at[c, pl.ds(s * 3120, 3120), :])

    @pl.when(s == 0)
    def _():
        pltpu.sync_copy(acc_sh.at[pl.ds(NSC * 3120, 80), :],
                        agg_h.at[c, pl.ds(NSC * 3120, 80), :])

    pltpu.sync_copy(eptab.at[pl.ds(0, B), :], ep_h.at[c, s])


@functools.cache
def _scatter_k():
    return pl.kernel(
        _scatter_body,
        out_type=(_sds((NCC, N, 32)), _sds((NCC, NSC, B, 32))),
        mesh=_mesh(),
        compiler_params=pltpu.CompilerParams(use_tc_tiling_on_sc=False),
        scratch_types=[
            pltpu.VMEM((C,), _i32),
            pltpu.VMEM((C, H), _f32),
            pltpu.VMEM((C, 32), _f32),
            pltpu.VMEM((C, 32), _f32),
            pltpu.VMEM((B + 8, 32), _f32),
            pltpu.VMEM_SHARED((N, 32), _f32),
        ])


def _readout_body(x_h, batch_h, sump_h, maxp_h,
                  idx_v, x_buf, idx40_v, x40_buf, sumtab, maxtab):
    c = lax.axis_index("c")
    s = lax.axis_index("s")
    w = s * NCC + c

    def init_tab(i, _):
        for q in range(4):
            sl = pl.ds(q * 16, 16)
            sumtab[i, sl] = jnp.zeros((16,), _f32)
            maxtab[i, sl] = jnp.full((16,), -jnp.inf, _f32)
        return 0
    lax.fori_loop(0, B + 8, init_tab, 0)

    def accgrp(ibuf, xbuf, g):
        bvec = ibuf[pl.ds(g * 16, 16)]
        for j in range(16):
            b = bvec[j]
            for q in range(4):
                sl = pl.ds(q * 16, 16)
                sumtab[b, sl] = sumtab[b, sl] + xbuf[g * 16 + j, sl]
                maxtab[b, sl] = jnp.maximum(maxtab[b, sl], xbuf[g * 16 + j, sl])

    n = 12 + jnp.where(s < 3, 1, 0)

    def body(i, _):
        base = c * NHALF + (s * 12 + jnp.minimum(s, 3) + i) * C
        pltpu.sync_copy(x_h.at[pl.ds(base, C), :], x_buf)
        pltpu.sync_copy(batch_h.at[pl.ds(base, C)], idx_v)

        def rb(g, _):
            accgrp(idx_v, x_buf, g)
            return 0
        lax.fori_loop(0, C // 16, rb, 0)
        return 0
    lax.fori_loop(0, n, body, 0)

    @pl.when(s == NSC - 1)
    def _():
        base = c * NHALF + HCH * C
        pltpu.sync_copy(x_h.at[pl.ds(base, HREM), :],
                        x40_buf.at[pl.ds(0, HREM), :])
        pltpu.sync_copy(batch_h.at[pl.ds(base, HREM)],
                        idx40_v.at[pl.ds(0, HREM)])
        lane = lax.iota(_i32, 16)
        tail = idx40_v[pl.ds(32, 16)]
        idx40_v[pl.ds(32, 16)] = jnp.where(lane < 8, tail, B)

        def rb40(g, _):
            accgrp(idx40_v, x40_buf, g)
            return 0
        lax.fori_loop(0, 3, rb40, 0)

    pltpu.sync_copy(sumtab.at[pl.ds(0, B), :], sump_h.at[w])
    pltpu.sync_copy(maxtab.at[pl.ds(0, B), :], maxp_h.at[w])


@functools.cache
def _readout_sc_k():
    return pl.kernel(
        _readout_body,
        out_type=(_sds((NW, B, H)), _sds((NW, B, H))),
        mesh=_mesh(),
        compiler_params=pltpu.CompilerParams(use_tc_tiling_on_sc=False),
        scratch_types=[
            pltpu.VMEM((C,), _i32), pltpu.VMEM((C, H), _f32),
            pltpu.VMEM((48,), _i32), pltpu.VMEM((48, H), _f32),
            pltpu.VMEM((B + 8, H), _f32), pltpu.VMEM((B + 8, H), _f32),
        ])


def _ln(t, g, b):
    m = jnp.mean(t, axis=-1, keepdims=True)
    v = jnp.mean((t - m) ** 2, axis=-1, keepdims=True)
    return (t - m) * lax.rsqrt(v + 1e-5) * g + b


def _silu(t):
    return t * jax.nn.sigmoid(t)


def _mm(a, b):
    return jnp.dot(a, b, preferred_element_type=_f32)


def _u0_body(af, seW, seb, seg, sebe, wcat, u0_o, ut_o):
    u0 = _silu(_ln(_mm(af[...], seW[...]) + seb[...], seg[...], sebe[...]))
    u0_o[...] = u0
    ut_o[...] = _mm(u0, wcat[...])


def _tc_u0(af, seW, seb, seg, sebe, wcat):
    return pl.pallas_call(
        _u0_body,
        out_shape=(_sds((B, H)), _sds((B, 2 * H))),
    )(af, seW, seb, seg, sebe, wcat)


def _state_body_mk(has_ut):
    def body(*refs):
        if has_ut:
            (u, ep, ec, sA, sB, sb1, sg, sbe, sW2, sb2, wcat,
             u_o, ut_o) = refs
        else:
            u, ep, ec, sA, sB, sb1, sg, sbe, sW2, sb2, u_o = refs
        ept = jnp.concatenate([jnp.sum(ep[0], axis=0),
                               jnp.sum(ep[1], axis=0)], axis=-1)
        ecs = jnp.sum(ec[...], axis=(0, 1))[:, 0:1]
        epm = ept / jnp.maximum(ecs, 1.0)
        t = _mm(u[...], sA[...]) + _mm(epm, sB[...]) + sb1[...]
        h = _silu(_ln(t, sg[...], sbe[...]))
        un = u[...] + _mm(h, sW2[...]) + sb2[...]
        u_o[...] = un
        if has_ut:
            ut_o[...] = _mm(un, wcat[...])
    return body


def _tc_state(u, ep, ec, sA, sB, sb1, sg, sbe, sW2, sb2, wcat=None):
    if wcat is not None:
        return pl.pallas_call(
            _state_body_mk(True),
            out_shape=(_sds((B, H)), _sds((B, 2 * H))),
        )(u, ep, ec, sA, sB, sb1, sg, sbe, sW2, sb2, wcat)
    return pl.pallas_call(
        _state_body_mk(False),
        out_shape=_sds((B, H)),
    )(u, ep, ec, sA, sB, sb1, sg, sbe, sW2, sb2)


BE = 8000


def _edge_body_mk(enc):
    def body(*refs):
        if enc:
            (ein, G, encW, encb, encg, encbe,
             We, eb1, eg, ebe, W2, eb2, enew_o, eout_o) = refs
            e0 = _silu(_ln(_mm(ein[...], encW[...]) + encb[...],
                           encg[...], encbe[...]))
        else:
            ein, G, We, eb1, eg, ebe, W2, eb2, enew_o, eout_o = refs
            e0 = ein[...]
        t = _mm(e0, We[...]) + G[...] + eb1[...]
        h = _silu(_ln(t, eg[...], ebe[...]))
        en = _mm(h, W2[...]) + eb2[...]
        enew_o[...] = en
        eout_o[...] = e0 + en
    return body


def _full(shape):
    return pl.BlockSpec(shape, lambda i: (0,) * len(shape))


def _tc_edge(ein, G, enc_w, We, eb1, eg, ebe, W2, eb2):
    enc = enc_w is not None
    din = ein.shape[1]
    ins = [pl.BlockSpec((BE, din), lambda i: (i, 0)),
           pl.BlockSpec((BE, H), lambda i: (i, 0))]
    args = [ein, G]
    if enc:
        ins += [_full(enc_w[0].shape), _full((1, H)), _full((1, H)),
                _full((1, H))]
        args += list(enc_w)
    ins += [_full((H, H)), _full((1, H)), _full((1, H)), _full((1, H)),
            _full((H, H)), _full((1, H))]
    args += [We, eb1, eg, ebe, W2, eb2]
    return pl.pallas_call(
        _edge_body_mk(enc),
        grid=(E // BE,),
        in_specs=ins,
        out_specs=(pl.BlockSpec((BE, H), lambda i: (i, 0)),
                   pl.BlockSpec((BE, H), lambda i: (i, 0))),
        out_shape=(_sds((E, H)), _sds((E, H))),
    )(*args)


BN = 2000


def _encx_body(xr, ub, neW, neb, neg, nebe, Wxi, Wxj, x_o, t12_o):
    x0 = _silu(_ln(_mm(xr[...], neW[...]) + neb[...], neg[...], nebe[...]))
    x_o[...] = x0
    t12_o[...] = jnp.concatenate(
        [_mm(x0, Wxi[...]) + ub[:, 0:H], _mm(x0, Wxj[...])], axis=-1)


def _tc_encx(xr, ub2, neW, neb, neg, nebe, Wxi, Wxj):
    return pl.pallas_call(
        _encx_body,
        grid=(N // BN,),
        in_specs=[pl.BlockSpec((BN, 16), lambda i: (i, 0)),
                  pl.BlockSpec((BN, 2 * H), lambda i: (i, 0)),
                  _full((16, H)), _full((1, H)), _full((1, H)), _full((1, H)),
                  _full((H, H)), _full((H, H))],
        out_specs=(pl.BlockSpec((BN, H), lambda i: (i, 0)),
                   pl.BlockSpec((BN, 2 * H), lambda i: (i, 0))),
        out_shape=(_sds((N, H)), _sds((N, 2 * H))),
    )(xr, ub2, neW, neb, neg, nebe, Wxi, Wxj)


def _node_body_mk(tables):
    def body(*refs):
        if tables:
            (xin, agg0, agg1, cnt0, cnt1, ub, ubn,
             nA, nB, nb1, ng, nbe, nW2, nb2, Wxi, Wxj,
             x_o, t12_o) = refs
        else:
            (xin, agg0, agg1, cnt0, cnt1, ub,
             nA, nB, nb1, ng, nbe, nW2, nb2, x_o) = refs
        agg = jnp.concatenate([agg0[0], agg1[0]], axis=-1)
        cnt = cnt0[0, :, 0:1] + cnt1[0, :, 0:1]
        aggm = agg / jnp.maximum(cnt, 1.0)
        t = _mm(xin[...], nA[...]) + _mm(aggm, nB[...]) + ub[:, H:2 * H] \
            + nb1[...]
        h = _silu(_ln(t, ng[...], nbe[...]))
        xn = xin[...] + _mm(h, nW2[...]) + nb2[...]
        x_o[...] = xn
        if tables:
            t12_o[...] = jnp.concatenate(
                [_mm(xn, Wxi[...]) + ubn[:, 0:H], _mm(xn, Wxj[...])], axis=-1)
    return body


def _tc_node(xin, agg, cnt, ub2, nA, nB, nb1, ng, nbe, nW2, nb2,
             ub2_next=None, Wxi=None, Wxj=None):
    tables = ub2_next is not None
    ins = [pl.BlockSpec((BN, H), lambda i: (i, 0)),
           pl.BlockSpec((1, BN, 32), lambda i: (0, i, 0)),
           pl.BlockSpec((1, BN, 32), lambda i: (1, i, 0)),
           pl.BlockSpec((1, BN, 16), lambda i: (0, i, 0)),
           pl.BlockSpec((1, BN, 16), lambda i: (1, i, 0)),
           pl.BlockSpec((BN, 2 * H), lambda i: (i, 0))]
    args = [xin, agg, agg, cnt, cnt, ub2]
    wspecs = [_full((H, H)), _full((H, H)), _full((1, H)), _full((1, H)),
              _full((1, H)), _full((H, H)), _full((1, H))]
    if tables:
        ins += [pl.BlockSpec((BN, 2 * H), lambda i: (i, 0))]
        ins += wspecs + [_full((H, H)), _full((H, H))]
        args += [ub2_next, nA, nB, nb1, ng, nbe, nW2, nb2, Wxi, Wxj]
        return pl.pallas_call(
            _node_body_mk(True),
            grid=(N // BN,),
            in_specs=ins,
            out_specs=(pl.BlockSpec((BN, H), lambda i: (i, 0)),
                       pl.BlockSpec((BN, 2 * H), lambda i: (i, 0))),
            out_shape=(_sds((N, H)), _sds((N, 2 * H))),
        )(*args)
    ins += wspecs
    args += [nA, nB, nb1, ng, nbe, nW2, nb2]
    return pl.pallas_call(
        _node_body_mk(False),
        grid=(N // BN,),
        in_specs=ins,
        out_specs=pl.BlockSpec((BN, H), lambda i: (i, 0)),
        out_shape=_sds((N, H)),
    )(*args)


def _final_body(sp, mp, xc, u, o13, o2, o4, ob1, og, obe, oW2, ob2, oW3, ob3,
                out_o):
    xs = jnp.sum(sp[...], axis=0)
    xmax = jnp.max(mp[...], axis=0)
    xct = jnp.sum(xc[...], axis=(0, 1))[:, 0:1] * (1.0 / NCC)
    xm = xs / jnp.maximum(xct, 1.0)
    t = _mm(xm, o13[...]) + _mm(xmax, o2[...]) + _mm(u[...], o4[...]) + ob1[...]
    h = _silu(_ln(t, og[...], obe[...]))
    h = _silu(_mm(h, oW2[...]) + ob2[...])
    out_o[...] = _mm(h, oW3[...]) + ob3[...]


def _tc_final(sp, mp, xc, u, o13, o2, o4, ob1, og, obe, oW2, ob2, oW3, ob3):
    return pl.pallas_call(
        _final_body,
        out_shape=_sds((B, 1)),
    )(sp, mp, xc, u, o13, o2, o4, ob1, og, obe, oW2, ob2, oW3, ob3)


def kernel(x, edge_attr, additional_features, params, edge_index, batch):
    p = params
    row = edge_index[0]
    col = edge_index[1]

    def r1(v):
        return v.reshape(1, -1)

    L = []
    for lp in p['layers']:
        We = lp['eW1'][0:H]
        Wxi = lp['eW1'][H:2 * H]
        Wxj = lp['eW1'][2 * H:3 * H]
        Wu = lp['eW1'][3 * H:]
        nA = lp['nW1'][0:H]
        nB = lp['nW1'][H:2 * H]
        nC = lp['nW1'][2 * H:]
        sA = lp['sW1'][0:H]
        sB = lp['sW1'][H:]
        wcat = jnp.concatenate([Wu, nC], axis=1)
        L.append(dict(
            We=We, Wxi=Wxi, Wxj=Wxj, wcat=wcat,
            eb1=r1(lp['eb1']), eg=r1(lp['eg']), ebe=r1(lp['ebe']),
            eW2=lp['eW2'], eb2=r1(lp['eb2']),
            nA=nA, nB=nB, nb1=r1(lp['nb1']), ng=r1(lp['ng']), nbe=r1(lp['nbe']),
            nW2=lp['nW2'], nb2=r1(lp['nb2']),
            sA=sA, sB=sB, sb1=r1(lp['sb1']), sg=r1(lp['sg']), sbe=r1(lp['sbe']),
            sW2=lp['sW2'], sb2=r1(lp['sb2']),
        ))

    o13 = p['oW1'][0:H] + p['oW1'][2 * H:3 * H]
    o2 = p['oW1'][H:2 * H]
    o4 = p['oW1'][3 * H:]

    cnt, ec, xc = _prep_k()(row, batch)

    u, ut = _tc_u0(additional_features, p['se_W'], r1(p['se_b']),
                   r1(p['se_g']), r1(p['se_be']), L[0]['wcat'])
    ub2 = _gatherb_k()(batch, ut)
    xcur, T12 = _tc_encx(x, ub2, p['ne_W'], r1(p['ne_b']),
                         r1(p['ne_g']), r1(p['ne_be']),
                         L[0]['Wxi'], L[0]['Wxj'])

    ecur = edge_attr
    enc_w = (p['ee_W'], r1(p['ee_b']), r1(p['ee_g']), r1(p['ee_be']))

    for li, lw in enumerate(L):
        last = li == len(L) - 1
        G = _gathere_k()(row, col, T12)
        e_new, ecur = _tc_edge(ecur, G, enc_w if li == 0 else None,
                               lw['We'], lw['eb1'], lw['eg'], lw['ebe'],
                               lw['eW2'], lw['eb2'])
        agg, ep = _scatter_k()(e_new, row, batch)
        if not last:
            u, ut = _tc_state(u, ep, ec, lw['sA'], lw['sB'], lw['sb1'],
                              lw['sg'], lw['sbe'], lw['sW2'], lw['sb2'],
                              L[li + 1]['wcat'])
            ub2_next = _gatherb_k()(batch, ut)
            xcur, T12 = _tc_node(xcur, agg, cnt, ub2,
                                 lw['nA'], lw['nB'], lw['nb1'], lw['ng'],
                                 lw['nbe'], lw['nW2'], lw['nb2'],
                                 ub2_next, L[li + 1]['Wxi'],
                                 L[li + 1]['Wxj'])
            ub2 = ub2_next
        else:
            u = _tc_state(u, ep, ec, lw['sA'], lw['sB'], lw['sb1'],
                          lw['sg'], lw['sbe'], lw['sW2'], lw['sb2'])
            xcur = _tc_node(xcur, agg, cnt, ub2,
                            lw['nA'], lw['nB'], lw['nb1'], lw['ng'],
                            lw['nbe'], lw['nW2'], lw['nb2'])

    sump, maxp = _readout_sc_k()(xcur, batch)
    return _tc_final(sump, maxp, xc, u, o13, o2, o4,
                     r1(p['ob1']), r1(p['og']), r1(p['obe']),
                     p['oW2'], r1(p['ob2']), p['oW3'], r1(p['ob3']))


# speedup vs baseline: 3.9530x; 1.3287x over previous
import functools

import jax
import jax.numpy as jnp
from jax import lax
from jax.experimental import pallas as pl
from jax.experimental.pallas import tpu as pltpu
from jax.experimental.pallas import tpu_sc as plsc

N = 50000
E = 800000
B = 256
H = 64
NHALF = N // 2
C = 128
NCC = 2
NSC = 16
NW = NCC * NSC
ECH = E // C
ECH_HALF = ECH // 2
NCH = N // C
NREM = N - NCH * C
HCH = NHALF // C
HREM = NHALF - HCH * C


@functools.cache
def _mesh():
    return plsc.VectorSubcoreMesh(
        core_axis_name="c", subcore_axis_name="s",
        num_cores=NCC, num_subcores=NSC)


_f32 = jnp.float32
_i32 = jnp.int32


def _sds(shape, dtype=_f32):
    return jax.ShapeDtypeStruct(shape, dtype)


def _prep_body(row_h, batch_h, cnt_h, ec_h, xc_h,
               idx_v, nbuf_v, ones_v, zb_v, ectab, xctab, cnt_sh):
    c = lax.axis_index("c")
    s = lax.axis_index("s")

    def init_ones(i, _):
        ones_v[i, :] = jnp.ones((16,), _f32)
        return 0
    lax.fori_loop(0, C, init_ones, 0)

    def init_zb(i, _):
        zb_v[i, :] = jnp.zeros((16,), _f32)
        return 0
    lax.fori_loop(0, 624, init_zb, 0)

    def init_tabs(i, _):
        ectab[i, :] = jnp.zeros((16,), _f32)
        xctab[i, :] = jnp.zeros((16,), _f32)
        return 0
    lax.fori_loop(0, B + 8, init_tabs, 0)

    def zloop(i, _):
        pltpu.sync_copy(zb_v, cnt_sh.at[pl.ds(s * 3120 + i * 624, 624), :])
        return 0
    lax.fori_loop(0, 5, zloop, 0)

    @pl.when(s == 0)
    def _():
        pltpu.sync_copy(zb_v.at[pl.ds(0, 80), :],
                        cnt_sh.at[pl.ds(NSC * 3120, 80), :])

    plsc.subcore_barrier()

    ne = 195 + jnp.where(s < 5, 1, 0)

    def ebody(i, _):
        cb = (c * ECH_HALF + s * 195 + jnp.minimum(s, 5) + i) * C
        pltpu.sync_copy(row_h.at[pl.ds(cb, C)], idx_v)
        pltpu.sync_copy(ones_v, cnt_sh.at[idx_v], add=True)
        return 0
    lax.fori_loop(0, ne, ebody, 0)

    plsc.subcore_barrier()

    def accgrp(g):
        bvec = idx_v[pl.ds(g * 16, 16)]
        for j in range(16):
            b = bvec[j]
            ectab[b, :] = ectab[b, :] + nbuf_v[g * 16 + j, :]
            xctab[b, :] = xctab[b, :] + jnp.ones((16,), _f32)

    nn = 24 + jnp.where(s < 6, 1, 0)

    def nbody(i, _):
        nb = (s * 24 + jnp.minimum(s, 6) + i) * C
        pltpu.sync_copy(batch_h.at[pl.ds(nb, C)], idx_v)
        pltpu.sync_copy(cnt_sh.at[pl.ds(nb, C), :], nbuf_v)

        def gb_(g, _):
            accgrp(g)
            return 0
        lax.fori_loop(0, C // 16, gb_, 0)
        return 0
    lax.fori_loop(0, nn, nbody, 0)

    @pl.when(s == NSC - 1)
    def _():
        nb = NCH * C
        pltpu.sync_copy(batch_h.at[pl.ds(nb, NREM)], idx_v.at[pl.ds(0, NREM)])
        pltpu.sync_copy(cnt_sh.at[pl.ds(nb, NREM), :],
                        nbuf_v.at[pl.ds(0, NREM), :])

        def gb_(g, _):
            accgrp(g)
            return 0
        lax.fori_loop(0, NREM // 16, gb_, 0)

    pltpu.sync_copy(cnt_sh.at[pl.ds(s * 3120, 3120), :],
                    cnt_h.at[c, pl.ds(s * 3120, 3120), :])

    @pl.when(s == 0)
    def _():
        pltpu.sync_copy(cnt_sh.at[pl.ds(NSC * 3120, 80), :],
                        cnt_h.at[c, pl.ds(NSC * 3120, 80), :])

    pltpu.sync_copy(ectab.at[pl.ds(0, B), :], ec_h.at[c, s])
    pltpu.sync_copy(xctab.at[pl.ds(0, B), :], xc_h.at[c, s])


@functools.cache
def _prep_k():
    return pl.kernel(
        _prep_body,
        out_type=(_sds((NCC, N, 16)), _sds((NCC, NSC, B, 16)),
                  _sds((NCC, NSC, B, 16))),
        mesh=_mesh(),
        compiler_params=pltpu.CompilerParams(use_tc_tiling_on_sc=False),
        scratch_types=[
            pltpu.VMEM((C,), _i32),
            pltpu.VMEM((C, 16), _f32),
            pltpu.VMEM((C, 16), _f32),
            pltpu.VMEM((624, 16), _f32),
            pltpu.VMEM((B + 8, 16), _f32),
            pltpu.VMEM((B + 8, 16), _f32),
            pltpu.VMEM_SHARED((N, 16), _f32),
        ])


def _gatherb_body(batch_h, ut_h, out_h, idx_v, rows_v, idx40_v, rows40_v, sem):
    c = lax.axis_index("c")
    s = lax.axis_index("s")
    w = s * NCC + c
    n = 12 + jnp.where(w < 6, 1, 0)

    def body(i, _):
        cb = (w * 12 + jnp.minimum(w, 6) + i) * C
        pltpu.sync_copy(batch_h.at[pl.ds(cb, C)], idx_v)
        pltpu.async_copy(ut_h.at[idx_v], rows_v, sem).wait()
        pltpu.sync_copy(rows_v, out_h.at[pl.ds(cb, C), :])
        return 0
    lax.fori_loop(0, n, body, 0)

    @pl.when(w == NW - 1)
    def _():
        pltpu.sync_copy(batch_h.at[pl.ds(NCH * C, NREM)], idx40_v)
        pltpu.async_copy(ut_h.at[idx40_v], rows40_v, sem).wait()
        pltpu.sync_copy(rows40_v, out_h.at[pl.ds(NCH * C, NREM), :])


@functools.cache
def _gatherb_k():
    return pl.kernel(
        _gatherb_body,
        out_type=_sds((N, 2 * H)),
        mesh=_mesh(),
        compiler_params=pltpu.CompilerParams(use_tc_tiling_on_sc=False),
        scratch_types=[
            pltpu.VMEM((C,), _i32), pltpu.VMEM((C, 2 * H), _f32),
            pltpu.VMEM((NREM,), _i32), pltpu.VMEM((NREM, 2 * H), _f32),
            pltpu.SemaphoreType.DMA,
        ])


def _gathere_body(row_h, col_h, t12_h, g_h,
                  ridx0, cidx0, ridx1, cidx1, a0, b0, a1, b1, g_buf,
                  sem0, sem1):
    c = lax.axis_index("c")
    s = lax.axis_index("s")
    w = s * NCC + c
    n = 195 + jnp.where(w < 10, 1, 0)

    def base(i):
        return (w * 195 + jnp.minimum(w, 10) + i) * C

    def load_issue(i, ridx, cidx, a, b, sem):
        cb = base(i)
        pltpu.sync_copy(row_h.at[pl.ds(cb, C)], ridx)
        pltpu.sync_copy(col_h.at[pl.ds(cb, C)], cidx)
        pltpu.async_copy(t12_h.at[ridx], a, sem)
        pltpu.async_copy(t12_h.at[cidx], b, sem)

    def consume(i, ridx, cidx, a, b, sem):
        pltpu.make_async_copy(t12_h.at[ridx], a, sem).wait()
        pltpu.make_async_copy(t12_h.at[cidx], b, sem).wait()

        def abody(r, _):
            for q in range(4):
                sl = pl.ds(q * 16, 16)
                g_buf[r, sl] = a[r, sl] + b[r, pl.ds(H + q * 16, 16)]
            return 0
        lax.fori_loop(0, C, abody, 0)
        pltpu.sync_copy(g_buf, g_h.at[pl.ds(base(i), C), :])

    load_issue(0, ridx0, cidx0, a0, b0, sem0)

    def body(i, _):
        even = i % 2 == 0

        @pl.when((i + 1 < n) & even)
        def _():
            load_issue(i + 1, ridx1, cidx1, a1, b1, sem1)

        @pl.when((i + 1 < n) & jnp.logical_not(even))
        def _():
            load_issue(i + 1, ridx0, cidx0, a0, b0, sem0)

        @pl.when(even)
        def _():
            consume(i, ridx0, cidx0, a0, b0, sem0)

        @pl.when(jnp.logical_not(even))
        def _():
            consume(i, ridx1, cidx1, a1, b1, sem1)
        return 0
    lax.fori_loop(0, n, body, 0)


@functools.cache
def _gathere_k():
    return pl.kernel(
        _gathere_body,
        out_type=_sds((E, H)),
        mesh=_mesh(),
        compiler_params=pltpu.CompilerParams(use_tc_tiling_on_sc=False),
        scratch_types=[
            pltpu.VMEM((C,), _i32), pltpu.VMEM((C,), _i32),
            pltpu.VMEM((C,), _i32), pltpu.VMEM((C,), _i32),
            pltpu.VMEM((C, 2 * H), _f32), pltpu.VMEM((C, 2 * H), _f32),
            pltpu.VMEM((C, 2 * H), _f32), pltpu.VMEM((C, 2 * H), _f32),
            pltpu.VMEM((C, H), _f32),
            pltpu.SemaphoreType.DMA, pltpu.SemaphoreType.DMA,
        ])


def _scatter_body(enew_h, row_h, batch_h, agg_h, ep_h,
                  idx_v, idx2_v, e_buf, e_buf2, cbuf, eptab,
                  sem0, sem1, acc_sh):
    nbuf_v = cbuf
    c = lax.axis_index("c")
    s = lax.axis_index("s")

    def init_cb(i, _):
        cbuf[i, pl.ds(0, 16)] = jnp.zeros((16,), _f32)
        cbuf[i, pl.ds(16, 16)] = jnp.zeros((16,), _f32)
        return 0
    lax.fori_loop(0, C, init_cb, 0)

    def init_tab(i, _):
        eptab[i, pl.ds(0, 16)] = jnp.zeros((16,), _f32)
        eptab[i, pl.ds(16, 16)] = jnp.zeros((16,), _f32)
        return 0
    lax.fori_loop(0, B + 8, init_tab, 0)

    def zloop(i, _):
        pltpu.sync_copy(cbuf, acc_sh.at[pl.ds(s * 3120 + i * C, C), :])
        return 0
    lax.fori_loop(0, 24, zloop, 0)

    pltpu.sync_copy(cbuf.at[pl.ds(0, 48), :],
                    acc_sh.at[pl.ds(s * 3120 + 24 * C, 48), :])

    @pl.when(s == 0)
    def _():
        pltpu.sync_copy(cbuf.at[pl.ds(0, 80), :],
                        acc_sh.at[pl.ds(NSC * 3120, 80), :])

    plsc.subcore_barrier()

    n = 390 + jnp.where(s < 10, 1, 0)

    def base(i):
        return (s * 390 + jnp.minimum(s, 10) + i) * C

    def load_issue(i, idxb, eb, sem):
        cb = base(i)
        pltpu.async_copy(enew_h.at[pl.ds(cb, C), :], eb, sem)
        pltpu.async_copy(row_h.at[pl.ds(cb, C)], idxb, sem)

    def consume(i, idxb, eb, sem):
        pltpu.make_async_copy(enew_h.at[pl.ds(base(i), C), :], eb, sem).wait()
        pltpu.make_async_copy(row_h.at[pl.ds(base(i), C)], idxb, sem).wait()

        @pl.when(c == 0)
        def _():
            def cp0(r, _):
                cbuf[r, pl.ds(0, 16)] = eb[r, pl.ds(0, 16)]
                cbuf[r, pl.ds(16, 16)] = eb[r, pl.ds(16, 16)]
                return 0
            lax.fori_loop(0, C, cp0, 0)

        @pl.when(c == 1)
        def _():
            def cp1(r, _):
                cbuf[r, pl.ds(0, 16)] = eb[r, pl.ds(32, 16)]
                cbuf[r, pl.ds(16, 16)] = eb[r, pl.ds(48, 16)]
                return 0
            lax.fori_loop(0, C, cp1, 0)

        pltpu.sync_copy(cbuf, acc_sh.at[idxb], add=True)

    load_issue(0, idx_v, e_buf, sem0)

    def body(i, _):
        even = i % 2 == 0

        @pl.when((i + 1 < n) & even)
        def _():
            load_issue(i + 1, idx2_v, e_buf2, sem1)

        @pl.when((i + 1 < n) & jnp.logical_not(even))
        def _():
            load_issue(i + 1, idx_v, e_buf, sem0)

        @pl.when(even)
        def _():
            consume(i, idx_v, e_buf, sem0)

        @pl.when(jnp.logical_not(even))
        def _():
            consume(i, idx2_v, e_buf2, sem1)
        return 0
    lax.fori_loop(0, n, body, 0)

    plsc.subcore_barrier()

    def accgrp(g):
        bvec = idx_v[pl.ds(g * 16, 16)]
        for j in range(16):
            b = bvec[j]
            for q in range(2):
                sl = pl.ds(q * 16, 16)
                eptab[b, sl] = eptab[b, sl] + nbuf_v[g * 16 + j, sl]

    nn = 24 + jnp.where(s < 6, 1, 0)

    def nbody(i, _):
        nb = (s * 24 + jnp.minimum(s, 6) + i) * C
        pltpu.sync_copy(batch_h.at[pl.ds(nb, C)], idx_v)
        pltpu.sync_copy(acc_sh.at[pl.ds(nb, C), :], nbuf_v)

        def gb_(g, _):
            accgrp(g)
            return 0
        lax.fori_loop(0, C // 16, gb_, 0)
        return 0
    lax.fori_loop(0, nn, nbody, 0)

    @pl.when(s == NSC - 1)
    def _():
        nb = NCH * C
        pltpu.sync_copy(batch_h.at[pl.ds(nb, NREM)], idx_v.at[pl.ds(0, NREM)])
        pltpu.sync_copy(acc_sh.at[pl.ds(nb, NREM), :],
                        nbuf_v.at[pl.ds(0, NREM), :])

        def gb_(g, _):
            accgrp(g)
            return 0
        lax.fori_loop(0, NREM // 16, gb_, 0)

    pltpu.sync_copy(acc_sh.at[pl.ds(s * 3120, 3120), :],
                    agg_h.at[c, pl.ds(s * 3120, 3120), :])

    @pl.when(s == 0)
    def _():
        pltpu.sync_copy(acc_sh.at[pl.ds(NSC * 3120, 80), :],
                        agg_h.at[c, pl.ds(NSC * 3120, 80), :])

    pltpu.sync_copy(eptab.at[pl.ds(0, B), :], ep_h.at[c, s])


@functools.cache
def _scatter_k():
    return pl.kernel(
        _scatter_body,
        out_type=(_sds((NCC, N, 32)), _sds((NCC, NSC, B, 32))),
        mesh=_mesh(),
        compiler_params=pltpu.CompilerParams(use_tc_tiling_on_sc=False),
        scratch_types=[
            pltpu.VMEM((C,), _i32), pltpu.VMEM((C,), _i32),
            pltpu.VMEM((C, H), _f32), pltpu.VMEM((C, H), _f32),
            pltpu.VMEM((C, 32), _f32),
            pltpu.VMEM((B + 8, 32), _f32),
            pltpu.SemaphoreType.DMA, pltpu.SemaphoreType.DMA,
            pltpu.VMEM_SHARED((N, 32), _f32),
        ])


def _readout_body(x_h, batch_h, sump_h, maxp_h,
                  idx_v, x_buf, idx40_v, x40_buf, sumtab, maxtab):
    c = lax.axis_index("c")
    s = lax.axis_index("s")
    w = s * NCC + c

    def init_tab(i, _):
        for q in range(4):
            sl = pl.ds(q * 16, 16)
            sumtab[i, sl] = jnp.zeros((16,), _f32)
            maxtab[i, sl] = jnp.full((16,), -jnp.inf, _f32)
        return 0
    lax.fori_loop(0, B + 8, init_tab, 0)

    def accgrp(ibuf, xbuf, g):
        bvec = ibuf[pl.ds(g * 16, 16)]
        for j in range(16):
            b = bvec[j]
            for q in range(4):
                sl = pl.ds(q * 16, 16)
                sumtab[b, sl] = sumtab[b, sl] + xbuf[g * 16 + j, sl]
                maxtab[b, sl] = jnp.maximum(maxtab[b, sl], xbuf[g * 16 + j, sl])

    n = 12 + jnp.where(s < 3, 1, 0)

    def body(i, _):
        base = c * NHALF + (s * 12 + jnp.minimum(s, 3) + i) * C
        pltpu.sync_copy(x_h.at[pl.ds(base, C), :], x_buf)
        pltpu.sync_copy(batch_h.at[pl.ds(base, C)], idx_v)

        def rb(g, _):
            accgrp(idx_v, x_buf, g)
            return 0
        lax.fori_loop(0, C // 16, rb, 0)
        return 0
    lax.fori_loop(0, n, body, 0)

    @pl.when(s == NSC - 1)
    def _():
        base = c * NHALF + HCH * C
        pltpu.sync_copy(x_h.at[pl.ds(base, HREM), :],
                        x40_buf.at[pl.ds(0, HREM), :])
        pltpu.sync_copy(batch_h.at[pl.ds(base, HREM)],
                        idx40_v.at[pl.ds(0, HREM)])
        lane = lax.iota(_i32, 16)
        tail = idx40_v[pl.ds(32, 16)]
        idx40_v[pl.ds(32, 16)] = jnp.where(lane < 8, tail, B)

        def rb40(g, _):
            accgrp(idx40_v, x40_buf, g)
            return 0
        lax.fori_loop(0, 3, rb40, 0)

    pltpu.sync_copy(sumtab.at[pl.ds(0, B), :], sump_h.at[w])
    pltpu.sync_copy(maxtab.at[pl.ds(0, B), :], maxp_h.at[w])


@functools.cache
def _readout_sc_k():
    return pl.kernel(
        _readout_body,
        out_type=(_sds((NW, B, H)), _sds((NW, B, H))),
        mesh=_mesh(),
        compiler_params=pltpu.CompilerParams(use_tc_tiling_on_sc=False),
        scratch_types=[
            pltpu.VMEM((C,), _i32), pltpu.VMEM((C, H), _f32),
            pltpu.VMEM((48,), _i32), pltpu.VMEM((48, H), _f32),
            pltpu.VMEM((B + 8, H), _f32), pltpu.VMEM((B + 8, H), _f32),
        ])


def _ln(t, g, b):
    m = jnp.mean(t, axis=-1, keepdims=True)
    v = jnp.mean((t - m) ** 2, axis=-1, keepdims=True)
    return (t - m) * lax.rsqrt(v + 1e-5) * g + b


def _silu(t):
    return t * jax.nn.sigmoid(t)


def _mm(a, b):
    return jnp.dot(a, b, preferred_element_type=_f32)


def _u0_body(af, seW, seb, seg, sebe, wcat, u0_o, ut_o):
    u0 = _silu(_ln(_mm(af[...], seW[...]) + seb[...], seg[...], sebe[...]))
    u0_o[...] = u0
    ut_o[...] = _mm(u0, wcat[...])


def _tc_u0(af, seW, seb, seg, sebe, wcat):
    return pl.pallas_call(
        _u0_body,
        out_shape=(_sds((B, H)), _sds((B, 2 * H))),
    )(af, seW, seb, seg, sebe, wcat)


def _state_body_mk(has_ut):
    def body(*refs):
        if has_ut:
            (u, ep, ec, sA, sB, sb1, sg, sbe, sW2, sb2, wcat,
             u_o, ut_o) = refs
        else:
            u, ep, ec, sA, sB, sb1, sg, sbe, sW2, sb2, u_o = refs
        ept = jnp.concatenate([jnp.sum(ep[0], axis=0),
                               jnp.sum(ep[1], axis=0)], axis=-1)
        ecs = jnp.sum(ec[...], axis=(0, 1))[:, 0:1]
        epm = ept / jnp.maximum(ecs, 1.0)
        t = _mm(u[...], sA[...]) + _mm(epm, sB[...]) + sb1[...]
        h = _silu(_ln(t, sg[...], sbe[...]))
        un = u[...] + _mm(h, sW2[...]) + sb2[...]
        u_o[...] = un
        if has_ut:
            ut_o[...] = _mm(un, wcat[...])
    return body


def _tc_state(u, ep, ec, sA, sB, sb1, sg, sbe, sW2, sb2, wcat=None):
    if wcat is not None:
        return pl.pallas_call(
            _state_body_mk(True),
            out_shape=(_sds((B, H)), _sds((B, 2 * H))),
        )(u, ep, ec, sA, sB, sb1, sg, sbe, sW2, sb2, wcat)
    return pl.pallas_call(
        _state_body_mk(False),
        out_shape=_sds((B, H)),
    )(u, ep, ec, sA, sB, sb1, sg, sbe, sW2, sb2)


BE = 8000


def _edge_body_mk(enc):
    def body(*refs):
        if enc:
            (ein, G, encW, encb, encg, encbe,
             We, eb1, eg, ebe, W2, eb2, enew_o, eout_o) = refs
            e0 = _silu(_ln(_mm(ein[...], encW[...]) + encb[...],
                           encg[...], encbe[...]))
        else:
            ein, G, We, eb1, eg, ebe, W2, eb2, enew_o, eout_o = refs
            e0 = ein[...]
        t = _mm(e0, We[...]) + G[...] + eb1[...]
        h = _silu(_ln(t, eg[...], ebe[...]))
        en = _mm(h, W2[...]) + eb2[...]
        enew_o[...] = en
        eout_o[...] = e0 + en
    return body


def _full(shape):
    return pl.BlockSpec(shape, lambda i: (0,) * len(shape))


def _tc_edge(ein, G, enc_w, We, eb1, eg, ebe, W2, eb2):
    enc = enc_w is not None
    din = ein.shape[1]
    ins = [pl.BlockSpec((BE, din), lambda i: (i, 0)),
           pl.BlockSpec((BE, H), lambda i: (i, 0))]
    args = [ein, G]
    if enc:
        ins += [_full(enc_w[0].shape), _full((1, H)), _full((1, H)),
                _full((1, H))]
        args += list(enc_w)
    ins += [_full((H, H)), _full((1, H)), _full((1, H)), _full((1, H)),
            _full((H, H)), _full((1, H))]
    args += [We, eb1, eg, ebe, W2, eb2]
    return pl.pallas_call(
        _edge_body_mk(enc),
        grid=(E // BE,),
        in_specs=ins,
        out_specs=(pl.BlockSpec((BE, H), lambda i: (i, 0)),
                   pl.BlockSpec((BE, H), lambda i: (i, 0))),
        out_shape=(_sds((E, H)), _sds((E, H))),
    )(*args)


BN = 2000


def _encx_body(xr, ub, neW, neb, neg, nebe, Wxi, Wxj, x_o, t12_o):
    x0 = _silu(_ln(_mm(xr[...], neW[...]) + neb[...], neg[...], nebe[...]))
    x_o[...] = x0
    t12_o[...] = jnp.concatenate(
        [_mm(x0, Wxi[...]) + ub[:, 0:H], _mm(x0, Wxj[...])], axis=-1)


def _tc_encx(xr, ub2, neW, neb, neg, nebe, Wxi, Wxj):
    return pl.pallas_call(
        _encx_body,
        grid=(N // BN,),
        in_specs=[pl.BlockSpec((BN, 16), lambda i: (i, 0)),
                  pl.BlockSpec((BN, 2 * H), lambda i: (i, 0)),
                  _full((16, H)), _full((1, H)), _full((1, H)), _full((1, H)),
                  _full((H, H)), _full((H, H))],
        out_specs=(pl.BlockSpec((BN, H), lambda i: (i, 0)),
                   pl.BlockSpec((BN, 2 * H), lambda i: (i, 0))),
        out_shape=(_sds((N, H)), _sds((N, 2 * H))),
    )(xr, ub2, neW, neb, neg, nebe, Wxi, Wxj)


def _node_body_mk(tables):
    def body(*refs):
        if tables:
            (xin, agg0, agg1, cnt0, cnt1, ub, ubn,
             nA, nB, nb1, ng, nbe, nW2, nb2, Wxi, Wxj,
             x_o, t12_o) = refs
        else:
            (xin, agg0, agg1, cnt0, cnt1, ub,
             nA, nB, nb1, ng, nbe, nW2, nb2, x_o) = refs
        agg = jnp.concatenate([agg0[0], agg1[0]], axis=-1)
        cnt = cnt0[0, :, 0:1] + cnt1[0, :, 0:1]
        aggm = agg / jnp.maximum(cnt, 1.0)
        t = _mm(xin[...], nA[...]) + _mm(aggm, nB[...]) + ub[:, H:2 * H] \
            + nb1[...]
        h = _silu(_ln(t, ng[...], nbe[...]))
        xn = xin[...] + _mm(h, nW2[...]) + nb2[...]
        x_o[...] = xn
        if tables:
            t12_o[...] = jnp.concatenate(
                [_mm(xn, Wxi[...]) + ubn[:, 0:H], _mm(xn, Wxj[...])], axis=-1)
    return body


def _tc_node(xin, agg, cnt, ub2, nA, nB, nb1, ng, nbe, nW2, nb2,
             ub2_next=None, Wxi=None, Wxj=None):
    tables = ub2_next is not None
    ins = [pl.BlockSpec((BN, H), lambda i: (i, 0)),
           pl.BlockSpec((1, BN, 32), lambda i: (0, i, 0)),
           pl.BlockSpec((1, BN, 32), lambda i: (1, i, 0)),
           pl.BlockSpec((1, BN, 16), lambda i: (0, i, 0)),
           pl.BlockSpec((1, BN, 16), lambda i: (1, i, 0)),
           pl.BlockSpec((BN, 2 * H), lambda i: (i, 0))]
    args = [xin, agg, agg, cnt, cnt, ub2]
    wspecs = [_full((H, H)), _full((H, H)), _full((1, H)), _full((1, H)),
              _full((1, H)), _full((H, H)), _full((1, H))]
    if tables:
        ins += [pl.BlockSpec((BN, 2 * H), lambda i: (i, 0))]
        ins += wspecs + [_full((H, H)), _full((H, H))]
        args += [ub2_next, nA, nB, nb1, ng, nbe, nW2, nb2, Wxi, Wxj]
        return pl.pallas_call(
            _node_body_mk(True),
            grid=(N // BN,),
            in_specs=ins,
            out_specs=(pl.BlockSpec((BN, H), lambda i: (i, 0)),
                       pl.BlockSpec((BN, 2 * H), lambda i: (i, 0))),
            out_shape=(_sds((N, H)), _sds((N, 2 * H))),
        )(*args)
    ins += wspecs
    args += [nA, nB, nb1, ng, nbe, nW2, nb2]
    return pl.pallas_call(
        _node_body_mk(False),
        grid=(N // BN,),
        in_specs=ins,
        out_specs=pl.BlockSpec((BN, H), lambda i: (i, 0)),
        out_shape=_sds((N, H)),
    )(*args)


def _final_body(sp, mp, xc, u, o13, o2, o4, ob1, og, obe, oW2, ob2, oW3, ob3,
                out_o):
    xs = jnp.sum(sp[...], axis=0)
    xmax = jnp.max(mp[...], axis=0)
    xct = jnp.sum(xc[...], axis=(0, 1))[:, 0:1] * (1.0 / NCC)
    xm = xs / jnp.maximum(xct, 1.0)
    t = _mm(xm, o13[...]) + _mm(xmax, o2[...]) + _mm(u[...], o4[...]) + ob1[...]
    h = _silu(_ln(t, og[...], obe[...]))
    h = _silu(_mm(h, oW2[...]) + ob2[...])
    out_o[...] = _mm(h, oW3[...]) + ob3[...]


def _tc_final(sp, mp, xc, u, o13, o2, o4, ob1, og, obe, oW2, ob2, oW3, ob3):
    return pl.pallas_call(
        _final_body,
        out_shape=_sds((B, 1)),
    )(sp, mp, xc, u, o13, o2, o4, ob1, og, obe, oW2, ob2, oW3, ob3)


def kernel(x, edge_attr, additional_features, params, edge_index, batch):
    p = params
    row = edge_index[0]
    col = edge_index[1]

    def r1(v):
        return v.reshape(1, -1)

    L = []
    for lp in p['layers']:
        We = lp['eW1'][0:H]
        Wxi = lp['eW1'][H:2 * H]
        Wxj = lp['eW1'][2 * H:3 * H]
        Wu = lp['eW1'][3 * H:]
        nA = lp['nW1'][0:H]
        nB = lp['nW1'][H:2 * H]
        nC = lp['nW1'][2 * H:]
        sA = lp['sW1'][0:H]
        sB = lp['sW1'][H:]
        wcat = jnp.concatenate([Wu, nC], axis=1)
        L.append(dict(
            We=We, Wxi=Wxi, Wxj=Wxj, wcat=wcat,
            eb1=r1(lp['eb1']), eg=r1(lp['eg']), ebe=r1(lp['ebe']),
            eW2=lp['eW2'], eb2=r1(lp['eb2']),
            nA=nA, nB=nB, nb1=r1(lp['nb1']), ng=r1(lp['ng']), nbe=r1(lp['nbe']),
            nW2=lp['nW2'], nb2=r1(lp['nb2']),
            sA=sA, sB=sB, sb1=r1(lp['sb1']), sg=r1(lp['sg']), sbe=r1(lp['sbe']),
            sW2=lp['sW2'], sb2=r1(lp['sb2']),
        ))

    o13 = p['oW1'][0:H] + p['oW1'][2 * H:3 * H]
    o2 = p['oW1'][H:2 * H]
    o4 = p['oW1'][3 * H:]

    cnt, ec, xc = _prep_k()(row, batch)

    u, ut = _tc_u0(additional_features, p['se_W'], r1(p['se_b']),
                   r1(p['se_g']), r1(p['se_be']), L[0]['wcat'])
    ub2 = _gatherb_k()(batch, ut)
    xcur, T12 = _tc_encx(x, ub2, p['ne_W'], r1(p['ne_b']),
                         r1(p['ne_g']), r1(p['ne_be']),
                         L[0]['Wxi'], L[0]['Wxj'])

    ecur = edge_attr
    enc_w = (p['ee_W'], r1(p['ee_b']), r1(p['ee_g']), r1(p['ee_be']))

    for li, lw in enumerate(L):
        last = li == len(L) - 1
        G = _gathere_k()(row, col, T12)
        e_new, ecur = _tc_edge(ecur, G, enc_w if li == 0 else None,
                               lw['We'], lw['eb1'], lw['eg'], lw['ebe'],
                               lw['eW2'], lw['eb2'])
        agg, ep = _scatter_k()(e_new, row, batch)
        if not last:
            u, ut = _tc_state(u, ep, ec, lw['sA'], lw['sB'], lw['sb1'],
                              lw['sg'], lw['sbe'], lw['sW2'], lw['sb2'],
                              L[li + 1]['wcat'])
            ub2_next = _gatherb_k()(batch, ut)
            xcur, T12 = _tc_node(xcur, agg, cnt, ub2,
                                 lw['nA'], lw['nB'], lw['nb1'], lw['ng'],
                                 lw['nbe'], lw['nW2'], lw['nb2'],
                                 ub2_next, L[li + 1]['Wxi'],
                                 L[li + 1]['Wxj'])
            ub2 = ub2_next
        else:
            u = _tc_state(u, ep, ec, lw['sA'], lw['sB'], lw['sb1'],
                          lw['sg'], lw['sbe'], lw['sW2'], lw['sb2'])
            xcur = _tc_node(xcur, agg, cnt, ub2,
                            lw['nA'], lw['nB'], lw['nb1'], lw['ng'],
                            lw['nbe'], lw['nW2'], lw['nb2'])

    sump, maxp = _readout_sc_k()(xcur, batch)
    return _tc_final(sump, maxp, xc, u, o13, o2, o4,
                     r1(p['ob1']), r1(p['og']), r1(p['obe']),
                     p['oW2'], r1(p['ob2']), p['oW3'], r1(p['ob3']))
